# Initial kernel scaffold; baseline (speedup 1.0000x reference)
import jax
import jax.numpy as jnp
from jax.experimental import pallas as pl


def kernel(params, noise_t, noise_i, input_ids):
    raise NotImplementedError("write your pallas kernel here")


# speedup vs baseline: 2.7702x; 2.7702x over previous
import functools

import jax
import jax.numpy as jnp
from jax import lax
from jax.experimental import pallas as pl
from jax.experimental.pallas import tpu as pltpu
from jax.experimental.pallas import tpu_sc as plsc

_B, _L, _H, _P, _E = 1024, 50, 128, 512, 4
_N = _B * _L
_NC, _NS = 2, 16
_NW = _NC * _NS
_PW = _N // _NW
_CH = 80
_NCHUNK = _PW // _CH
_T = 512


def _sc_gather(item_t, text_t, img_t, ids):
    @functools.partial(
        pl.kernel,
        mesh=plsc.VectorSubcoreMesh(core_axis_name="c", subcore_axis_name="s"),
        out_type=(
            jax.ShapeDtypeStruct((_N, _H), jnp.float32),
            jax.ShapeDtypeStruct((_N, _P), jnp.float32),
            jax.ShapeDtypeStruct((_N, _P), jnp.float32),
        ),
        scratch_types=(
            pltpu.VMEM((_CH,), jnp.int32),
            pltpu.VMEM((_CH, _H), jnp.float32),
            pltpu.VMEM((_CH, _P), jnp.float32),
            pltpu.SemaphoreType.DMA,
        ),
    )
    def gather_kernel(item_hbm, text_hbm, img_hbm, ids_hbm,
                      o_item, o_text, o_img, idx_v, buf_h, buf_p, sem):
        wid = lax.axis_index("s") * _NC + lax.axis_index("c")
        for k in range(_NCHUNK):
            base = wid * _PW + k * _CH
            sl = pl.ds(base, _CH)
            pltpu.sync_copy(ids_hbm.at[sl], idx_v)
            pltpu.async_copy(item_hbm.at[idx_v], buf_h, sem).wait()
            pltpu.sync_copy(buf_h, o_item.at[sl])
            pltpu.async_copy(text_hbm.at[idx_v], buf_p, sem).wait()
            pltpu.sync_copy(buf_p, o_text.at[sl])
            pltpu.async_copy(img_hbm.at[idx_v], buf_p, sem).wait()
            pltpu.sync_copy(buf_p, o_img.at[sl])

    return gather_kernel(item_t, text_t, img_t, ids)


def _dot(a, b):
    return lax.dot_general(a, b, (((1,), (0,)), ((), ())),
                           preferred_element_type=jnp.float32)


def _tc_body(text_r, img_r, item_r, nt_r, ni_r,
             ftw, ftb, fiw, fib,
             mtw, mtb, stw, stb, miw, mib, siw, sib,
             gw, gb, tew, teb, iew, ieb,
             fw, fb, fg, fbeta, out_r):
    def proj(x, w, b):
        y = _dot(x, w[...]) + b[...]
        nrm = jnp.sqrt(jnp.sum(y * y, axis=-1, keepdims=True))
        return y / jnp.maximum(nrm, 1e-12)

    te = proj(text_r[...], ftw, ftb)
    ie = proj(img_r[...], fiw, fib)

    t_z = _dot(te, mtw[...]) + mtb[...] + jnp.exp(_dot(te, stw[...]) + stb[...]) * nt_r[...]
    i_z = _dot(ie, miw[...]) + mib[...] + jnp.exp(_dot(ie, siw[...]) + sib[...]) * ni_r[...]

    def moe(z, ew, eb):
        logits = _dot(z, gw[...]) + gb[...]
        c = [logits[:, j:j + 1] for j in range(_E)]
        m = jnp.maximum(jnp.maximum(c[0], c[1]), jnp.maximum(c[2], c[3]))
        e = [jnp.exp(cj - m) for cj in c]
        w = []
        for j in range(_E):
            rank = jnp.zeros_like(e[j])
            for i in range(_E):
                if i == j:
                    continue
                beats = (e[i] >= e[j]) if i < j else (e[i] > e[j])
                rank = rank + beats.astype(jnp.float32)
            w.append(jnp.where(rank < 2.0, e[j], 0.0))
        s = w[0] + w[1] + w[2] + w[3]
        acc = (w[0] / s) * (_dot(z, ew[0]) + eb[0])
        for j in range(1, _E):
            acc = acc + (w[j] / s) * (_dot(z, ew[j]) + eb[j])
        return acc

    t_out = moe(t_z, tew, teb)
    i_out = moe(i_z, iew, ieb)

    f = _dot(t_out, fw[0]) + _dot(i_out, fw[1]) + fb[...]
    mu = jnp.mean(f, axis=-1, keepdims=True)
    d = f - mu
    v = jnp.mean(d * d, axis=-1, keepdims=True)
    ln = d / jnp.sqrt(v + 1e-5) * fg[...] + fbeta[...]
    out_r[...] = item_r[...] + jnp.maximum(ln, 0.0)


def _tc_specs_and_args(item_g, text_g, img_g, nt, ni, p):
    tok = lambda d: pl.BlockSpec((_T, d), lambda i: (i, 0))
    full = lambda *shape: pl.BlockSpec(shape, lambda i: (0,) * len(shape))
    r2 = lambda x: x.reshape(1, -1)
    args = (
        text_g, img_g, item_g, nt, ni,
        p["fc_text_w"], r2(p["fc_text_b"]), p["fc_img_w"], r2(p["fc_img_b"]),
        p["mu_t_w"], r2(p["mu_t_b"]), p["sg_t_w"], r2(p["sg_t_b"]),
        p["mu_i_w"], r2(p["mu_i_b"]), p["sg_i_w"], r2(p["sg_i_b"]),
        p["gate_w"], r2(p["gate_b"]),
        p["te_w"], p["te_b"].reshape(_E, 1, _H),
        p["ie_w"], p["ie_b"].reshape(_E, 1, _H),
        p["fus_w"].reshape(2, _H, _H), r2(p["fus_b"]),
        r2(p["fus_ln_g"]), r2(p["fus_ln_b"]),
    )
    in_specs = [
        tok(_P), tok(_P), tok(_H), tok(_H), tok(_H),
        full(_P, _H), full(1, _H), full(_P, _H), full(1, _H),
        full(_H, _H), full(1, _H), full(_H, _H), full(1, _H),
        full(_H, _H), full(1, _H), full(_H, _H), full(1, _H),
        full(_H, _E), full(1, _E),
        full(_E, _H, _H), full(_E, 1, _H),
        full(_E, _H, _H), full(_E, 1, _H),
        full(2, _H, _H), full(1, _H),
        full(1, _H), full(1, _H),
    ]
    return in_specs, args


def _tc_dense(item_g, text_g, img_g, nt, ni, p):
    in_specs, args = _tc_specs_and_args(item_g, text_g, img_g, nt, ni, p)
    return pl.pallas_call(
        _tc_body,
        grid=(_N // _T,),
        in_specs=in_specs,
        out_specs=pl.BlockSpec((_T, _H), lambda i: (i, 0)),
        out_shape=jax.ShapeDtypeStruct((_N, _H), jnp.float32),
        compiler_params=pltpu.CompilerParams(
            dimension_semantics=("arbitrary",),
        ),
    )(*args)


def kernel(params, noise_t, noise_i, input_ids):
    p = params
    ids = input_ids.reshape(-1).astype(jnp.int32)
    item_g, text_g, img_g = _sc_gather(
        p["item_table"], p["text_table"], p["img_table"], ids)
    nt = noise_t.reshape(_N, _H)
    ni = noise_i.reshape(_N, _H)
    out = _tc_dense(item_g, text_g, img_g, nt, ni, p)
    return out.reshape(_B, _L, _H)


# speedup vs baseline: 4.1229x; 1.4883x over previous
import functools

import jax
import jax.numpy as jnp
from jax import lax
from jax.experimental import pallas as pl
from jax.experimental.pallas import tpu as pltpu
from jax.experimental.pallas import tpu_sc as plsc

_B, _L, _H, _P, _E = 1024, 50, 128, 512, 4
_N = _B * _L
_NC, _NS = 2, 16
_NW = _NC * _NS
_PW = _N // _NW
_CH = 80
_NCHUNK = _PW // _CH
_T = 512


def _sc_gather(item_t, text_t, img_t, ids):
    @functools.partial(
        pl.kernel,
        mesh=plsc.VectorSubcoreMesh(core_axis_name="c", subcore_axis_name="s"),
        out_type=(
            jax.ShapeDtypeStruct((_N, _H), jnp.float32),
            jax.ShapeDtypeStruct((_N, _P), jnp.float32),
            jax.ShapeDtypeStruct((_N, _P), jnp.float32),
        ),
        scratch_types=(
            pltpu.VMEM((_CH,), jnp.int32),
            pltpu.VMEM((_CH, _H), jnp.float32),
            pltpu.VMEM((_CH, _P), jnp.float32),
            pltpu.SemaphoreType.DMA,
        ),
    )
    def gather_kernel(item_hbm, text_hbm, img_hbm, ids_hbm,
                      o_item, o_text, o_img, idx_v, buf_h, buf_p, sem):
        wid = lax.axis_index("s") * _NC + lax.axis_index("c")
        for k in range(_NCHUNK):
            base = wid * _PW + k * _CH
            sl = pl.ds(base, _CH)
            pltpu.sync_copy(ids_hbm.at[sl], idx_v)
            pltpu.async_copy(item_hbm.at[idx_v], buf_h, sem).wait()
            pltpu.sync_copy(buf_h, o_item.at[sl])
            pltpu.async_copy(text_hbm.at[idx_v], buf_p, sem).wait()
            pltpu.sync_copy(buf_p, o_text.at[sl])
            pltpu.async_copy(img_hbm.at[idx_v], buf_p, sem).wait()
            pltpu.sync_copy(buf_p, o_img.at[sl])

    return gather_kernel(item_t, text_t, img_t, ids)


def _dot(a, b):
    return lax.dot_general(a, b, (((1,), (0,)), ((), ())),
                           preferred_element_type=jnp.float32)


def _tc_body(text_r, img_r, item_r, nt_r, ni_r,
             ftw, ftb, fiw, fib,
             mtw, mtb, stw, stb, miw, mib, siw, sib,
             gw, gb, tew, teb, iew, ieb,
             fw, fb, fg, fbeta, out_r):
    def proj(x, w, b):
        y = _dot(x, w[...]) + b[...]
        nrm = jnp.sqrt(jnp.sum(y * y, axis=-1, keepdims=True))
        return y / jnp.maximum(nrm, 1e-12)

    te = proj(text_r[...], ftw, ftb)
    ie = proj(img_r[...], fiw, fib)

    t_z = _dot(te, mtw[...]) + mtb[...] + jnp.exp(_dot(te, stw[...]) + stb[...]) * nt_r[...]
    i_z = _dot(ie, miw[...]) + mib[...] + jnp.exp(_dot(ie, siw[...]) + sib[...]) * ni_r[...]

    jj = lax.broadcasted_iota(jnp.int32, (_E, _E * _H), 0)
    ll = lax.broadcasted_iota(jnp.int32, (_E, _E * _H), 1)
    ee = (jj == (ll >> 7)).astype(jnp.float32)
    neg = jnp.float32(-1e30)

    def moe(z, ewc, ebc):
        logits = _dot(z, gw[...]) + gb[...]
        lt = logits.T
        ii = lax.broadcasted_iota(jnp.int32, (_E, _T), 0)
        m1 = jnp.max(lt, axis=0, keepdims=True)
        a1 = jnp.min(jnp.where(lt == m1, ii, _E), axis=0, keepdims=True)
        msk = jnp.where(ii == a1, neg, lt)
        m2 = jnp.max(msk, axis=0, keepdims=True)
        a2 = jnp.min(jnp.where(msk == m2, ii, _E), axis=0, keepdims=True)
        keep = (ii == a1) | (ii == a2)
        e = jnp.exp(lt - m1)
        w = jnp.where(keep, e, 0.0)
        wn = w / jnp.sum(w, axis=0, keepdims=True)
        gx = lax.dot_general(wn, ee, (((0,), (0,)), ((), ())),
                             preferred_element_type=jnp.float32)
        y = (_dot(z, ewc[...]) + ebc[...]) * gx
        return (y[:, 0:_H] + y[:, _H:2 * _H]
                + y[:, 2 * _H:3 * _H] + y[:, 3 * _H:4 * _H])

    t_out = moe(t_z, tew, teb)
    i_out = moe(i_z, iew, ieb)

    f = _dot(t_out, fw[0]) + _dot(i_out, fw[1]) + fb[...]
    mu = jnp.mean(f, axis=-1, keepdims=True)
    d = f - mu
    v = jnp.mean(d * d, axis=-1, keepdims=True)
    ln = d / jnp.sqrt(v + 1e-5) * fg[...] + fbeta[...]
    out_r[...] = item_r[...] + jnp.maximum(ln, 0.0)


def _tc_specs_and_args(item_g, text_g, img_g, nt, ni, p):
    tok = lambda d: pl.BlockSpec((_T, d), lambda i: (i, 0))
    full = lambda *shape: pl.BlockSpec(shape, lambda i: (0,) * len(shape))
    r2 = lambda x: x.reshape(1, -1)
    args = (
        text_g, img_g, item_g, nt, ni,
        p["fc_text_w"], r2(p["fc_text_b"]), p["fc_img_w"], r2(p["fc_img_b"]),
        p["mu_t_w"], r2(p["mu_t_b"]), p["sg_t_w"], r2(p["sg_t_b"]),
        p["mu_i_w"], r2(p["mu_i_b"]), p["sg_i_w"], r2(p["sg_i_b"]),
        p["gate_w"], r2(p["gate_b"]),
        jnp.transpose(p["te_w"], (1, 0, 2)).reshape(_H, _E * _H),
        p["te_b"].reshape(1, _E * _H),
        jnp.transpose(p["ie_w"], (1, 0, 2)).reshape(_H, _E * _H),
        p["ie_b"].reshape(1, _E * _H),
        p["fus_w"].reshape(2, _H, _H), r2(p["fus_b"]),
        r2(p["fus_ln_g"]), r2(p["fus_ln_b"]),
    )
    in_specs = [
        tok(_P), tok(_P), tok(_H), tok(_H), tok(_H),
        full(_P, _H), full(1, _H), full(_P, _H), full(1, _H),
        full(_H, _H), full(1, _H), full(_H, _H), full(1, _H),
        full(_H, _H), full(1, _H), full(_H, _H), full(1, _H),
        full(_H, _E), full(1, _E),
        full(_H, _E * _H), full(1, _E * _H),
        full(_H, _E * _H), full(1, _E * _H),
        full(2, _H, _H), full(1, _H),
        full(1, _H), full(1, _H),
    ]
    return in_specs, args


def _tc_dense(item_g, text_g, img_g, nt, ni, p):
    in_specs, args = _tc_specs_and_args(item_g, text_g, img_g, nt, ni, p)
    return pl.pallas_call(
        _tc_body,
        grid=(_N // _T,),
        in_specs=in_specs,
        out_specs=pl.BlockSpec((_T, _H), lambda i: (i, 0)),
        out_shape=jax.ShapeDtypeStruct((_N, _H), jnp.float32),
        compiler_params=pltpu.CompilerParams(
            dimension_semantics=("arbitrary",),
        ),
    )(*args)


def kernel(params, noise_t, noise_i, input_ids):
    p = params
    ids = input_ids.reshape(-1).astype(jnp.int32)
    item_g, text_g, img_g = _sc_gather(
        p["item_table"], p["text_table"], p["img_table"], ids)
    nt = noise_t.reshape(_N, _H)
    ni = noise_i.reshape(_N, _H)
    out = _tc_dense(item_g, text_g, img_g, nt, ni, p)
    return out.reshape(_B, _L, _H)


# speedup vs baseline: 4.3864x; 1.0639x over previous
import functools

import jax
import jax.numpy as jnp
from jax import lax
from jax.experimental import pallas as pl
from jax.experimental.pallas import tpu as pltpu
from jax.experimental.pallas import tpu_sc as plsc

_B, _L, _H, _P, _E = 1024, 50, 128, 512, 4
_N = _B * _L
_NC, _NS = 2, 16
_NW = _NC * _NS
_PW = _N // _NW
_CH = 40
_NCHUNK = _PW // _CH
_T = 512


def _sc_gather(item_t, text_t, img_t, ids):
    @functools.partial(
        pl.kernel,
        mesh=plsc.VectorSubcoreMesh(core_axis_name="c", subcore_axis_name="s"),
        out_type=(
            jax.ShapeDtypeStruct((_N, _H), jnp.float32),
            jax.ShapeDtypeStruct((_N, _P), jnp.float32),
            jax.ShapeDtypeStruct((_N, _P), jnp.float32),
        ),
        scratch_types=(
            pltpu.VMEM((2, _CH), jnp.int32),
            pltpu.VMEM((2, _CH, _H), jnp.float32),
            pltpu.VMEM((2, _CH, _P), jnp.float32),
            pltpu.VMEM((2, _CH, _P), jnp.float32),
            pltpu.SemaphoreType.DMA,
            pltpu.SemaphoreType.DMA,
            pltpu.SemaphoreType.DMA,
            pltpu.SemaphoreType.DMA,
        ),
    )
    def gather_kernel(item_hbm, text_hbm, img_hbm, ids_hbm,
                      o_item, o_text, o_img,
                      idx_v, buf_h, buf_t, buf_i, g0, g1, w0, w1):
        wid = lax.axis_index("s") * _NC + lax.axis_index("c")
        gsem = (g0, g1)
        wsem = (w0, w1)
        gd = [None, None]
        wd = [None, None]

        def start(k):
            pr = k % 2
            if wd[pr] is not None:
                for d in wd[pr]:
                    d.wait()
                wd[pr] = None
            sl = pl.ds(wid * _PW + k * _CH, _CH)
            pltpu.sync_copy(ids_hbm.at[sl], idx_v.at[pr])
            gd[pr] = (
                pltpu.async_copy(item_hbm.at[idx_v.at[pr]], buf_h.at[pr], gsem[pr]),
                pltpu.async_copy(text_hbm.at[idx_v.at[pr]], buf_t.at[pr], gsem[pr]),
                pltpu.async_copy(img_hbm.at[idx_v.at[pr]], buf_i.at[pr], gsem[pr]),
            )

        def finish(k):
            pr = k % 2
            for d in gd[pr]:
                d.wait()
            gd[pr] = None
            sl = pl.ds(wid * _PW + k * _CH, _CH)
            wd[pr] = (
                pltpu.async_copy(buf_h.at[pr], o_item.at[sl], wsem[pr]),
                pltpu.async_copy(buf_t.at[pr], o_text.at[sl], wsem[pr]),
                pltpu.async_copy(buf_i.at[pr], o_img.at[sl], wsem[pr]),
            )

        start(0)
        for k in range(1, _NCHUNK):
            start(k)
            finish(k - 1)
        finish(_NCHUNK - 1)
        for pr in (0, 1):
            if wd[pr] is not None:
                for d in wd[pr]:
                    d.wait()

    return gather_kernel(item_t, text_t, img_t, ids)


def _dot(a, b):
    return lax.dot_general(a, b, (((1,), (0,)), ((), ())),
                           preferred_element_type=jnp.float32)


def _tc_body(text_r, img_r, item_r, nt_r, ni_r,
             ftw, ftb, fiw, fib,
             mtw, mtb, stw, stb, miw, mib, siw, sib,
             gw, gb, tew, teb, iew, ieb,
             fw, fb, fg, fbeta, out_r):
    def proj(x, w, b):
        y = _dot(x, w[...]) + b[...]
        nrm = jnp.sqrt(jnp.sum(y * y, axis=-1, keepdims=True))
        return y / jnp.maximum(nrm, 1e-12)

    te = proj(text_r[...], ftw, ftb)
    ie = proj(img_r[...], fiw, fib)

    t_z = _dot(te, mtw[...]) + mtb[...] + jnp.exp(_dot(te, stw[...]) + stb[...]) * nt_r[...]
    i_z = _dot(ie, miw[...]) + mib[...] + jnp.exp(_dot(ie, siw[...]) + sib[...]) * ni_r[...]

    jj = lax.broadcasted_iota(jnp.int32, (_E, _E * _H), 0)
    ll = lax.broadcasted_iota(jnp.int32, (_E, _E * _H), 1)
    ee = (jj == (ll >> 7)).astype(jnp.float32)
    neg = jnp.float32(-1e30)

    def moe(z, ewc, ebc):
        logits = _dot(z, gw[...]) + gb[...]
        lt = logits.T
        ii = lax.broadcasted_iota(jnp.int32, (_E, _T), 0)
        m1 = jnp.max(lt, axis=0, keepdims=True)
        a1 = jnp.min(jnp.where(lt == m1, ii, _E), axis=0, keepdims=True)
        msk = jnp.where(ii == a1, neg, lt)
        m2 = jnp.max(msk, axis=0, keepdims=True)
        a2 = jnp.min(jnp.where(msk == m2, ii, _E), axis=0, keepdims=True)
        keep = (ii == a1) | (ii == a2)
        e = jnp.exp(lt - m1)
        w = jnp.where(keep, e, 0.0)
        wn = w / jnp.sum(w, axis=0, keepdims=True)
        gx = lax.dot_general(wn, ee, (((0,), (0,)), ((), ())),
                             preferred_element_type=jnp.float32)
        y = (_dot(z, ewc[...]) + ebc[...]) * gx
        return (y[:, 0:_H] + y[:, _H:2 * _H]
                + y[:, 2 * _H:3 * _H] + y[:, 3 * _H:4 * _H])

    t_out = moe(t_z, tew, teb)
    i_out = moe(i_z, iew, ieb)

    f = _dot(t_out, fw[0]) + _dot(i_out, fw[1]) + fb[...]
    mu = jnp.mean(f, axis=-1, keepdims=True)
    d = f - mu
    v = jnp.mean(d * d, axis=-1, keepdims=True)
    ln = d / jnp.sqrt(v + 1e-5) * fg[...] + fbeta[...]
    out_r[...] = item_r[...] + jnp.maximum(ln, 0.0)


def _tc_specs_and_args(item_g, text_g, img_g, nt, ni, p):
    tok = lambda d: pl.BlockSpec((_T, d), lambda i: (i, 0))
    full = lambda *shape: pl.BlockSpec(shape, lambda i: (0,) * len(shape))
    r2 = lambda x: x.reshape(1, -1)
    args = (
        text_g, img_g, item_g, nt, ni,
        p["fc_text_w"], r2(p["fc_text_b"]), p["fc_img_w"], r2(p["fc_img_b"]),
        p["mu_t_w"], r2(p["mu_t_b"]), p["sg_t_w"], r2(p["sg_t_b"]),
        p["mu_i_w"], r2(p["mu_i_b"]), p["sg_i_w"], r2(p["sg_i_b"]),
        p["gate_w"], r2(p["gate_b"]),
        jnp.transpose(p["te_w"], (1, 0, 2)).reshape(_H, _E * _H),
        p["te_b"].reshape(1, _E * _H),
        jnp.transpose(p["ie_w"], (1, 0, 2)).reshape(_H, _E * _H),
        p["ie_b"].reshape(1, _E * _H),
        p["fus_w"].reshape(2, _H, _H), r2(p["fus_b"]),
        r2(p["fus_ln_g"]), r2(p["fus_ln_b"]),
    )
    in_specs = [
        tok(_P), tok(_P), tok(_H), tok(_H), tok(_H),
        full(_P, _H), full(1, _H), full(_P, _H), full(1, _H),
        full(_H, _H), full(1, _H), full(_H, _H), full(1, _H),
        full(_H, _H), full(1, _H), full(_H, _H), full(1, _H),
        full(_H, _E), full(1, _E),
        full(_H, _E * _H), full(1, _E * _H),
        full(_H, _E * _H), full(1, _E * _H),
        full(2, _H, _H), full(1, _H),
        full(1, _H), full(1, _H),
    ]
    return in_specs, args


def _tc_dense(item_g, text_g, img_g, nt, ni, p):
    in_specs, args = _tc_specs_and_args(item_g, text_g, img_g, nt, ni, p)
    return pl.pallas_call(
        _tc_body,
        grid=(_N // _T,),
        in_specs=in_specs,
        out_specs=pl.BlockSpec((_T, _H), lambda i: (i, 0)),
        out_shape=jax.ShapeDtypeStruct((_N, _H), jnp.float32),
        compiler_params=pltpu.CompilerParams(
            dimension_semantics=("arbitrary",),
        ),
    )(*args)


def kernel(params, noise_t, noise_i, input_ids):
    p = params
    ids = input_ids.reshape(-1).astype(jnp.int32)
    item_g, text_g, img_g = _sc_gather(
        p["item_table"], p["text_table"], p["img_table"], ids)
    nt = noise_t.reshape(_N, _H)
    ni = noise_i.reshape(_N, _H)
    out = _tc_dense(item_g, text_g, img_g, nt, ni, p)
    return out.reshape(_B, _L, _H)
